# Initial kernel scaffold; baseline (speedup 1.0000x reference)
import jax
import jax.numpy as jnp
from jax.experimental import pallas as pl


def kernel(h, codebook):
    raise NotImplementedError("write your pallas kernel here")


# speedup vs baseline: 1.5169x; 1.5169x over previous
import functools

import jax
import jax.numpy as jnp
from jax import lax
from jax.experimental import pallas as pl

B, T, D = 64, 1024, 32
K = 1024
TB = 1024


def _vq_body(h_ref, cb_ref, q_ref, idx_ref):
    h = h_ref[...]
    cb = cb_ref[...]
    xc = lax.dot_general(h, cb, (((1,), (1,)), ((), ())),
                         preferred_element_type=jnp.float32)
    x2 = jnp.sum(h * h, axis=1, keepdims=True)
    c2 = jnp.sum(cb * cb, axis=1)[None, :]
    dist = x2 + c2 - 2.0 * xc
    m = jnp.min(dist, axis=1, keepdims=True)
    iota = lax.broadcasted_iota(jnp.int32, (TB, K), 1)
    idx = jnp.min(jnp.where(dist <= m, iota, K), axis=1)
    idx_ref[0, 0, :] = idx
    onehot = (iota == idx[:, None]).astype(jnp.float32)
    q = lax.dot_general(onehot, cb, (((1,), (0,)), ((), ())),
                        preferred_element_type=jnp.float32)
    q_ref[...] = h + (q - h)


@jax.jit
def kernel(h, codebook):
    bsz, t, d = h.shape
    n = bsz * t
    grid = n // TB
    flat = h.reshape(n, d)
    q_flat, idx3 = pl.pallas_call(
        _vq_body,
        grid=(grid,),
        in_specs=[
            pl.BlockSpec((TB, d), lambda i: (i, 0)),
            pl.BlockSpec((K, d), lambda i: (0, 0)),
        ],
        out_specs=[
            pl.BlockSpec((TB, d), lambda i: (i, 0)),
            pl.BlockSpec((1, 1, TB), lambda i: (i, 0, 0)),
        ],
        out_shape=[
            jax.ShapeDtypeStruct((n, d), jnp.float32),
            jax.ShapeDtypeStruct((grid, 1, TB), jnp.int32),
        ],
    )(flat, codebook)
    return q_flat.reshape(bsz, t, d), idx3.reshape(bsz, t)


# speedup vs baseline: 2.2751x; 1.4999x over previous
import jax
import jax.numpy as jnp
from jax import lax
from jax.experimental import pallas as pl

B, T, D = 64, 1024, 32
K = 1024
TB = 1024


def _vq_body(h_ref, w1_ref, c2_ref, w2_ref, q_ref, idx_ref):
    h = h_ref[...]
    nxc = lax.dot_general(h, w1_ref[...], (((1,), (0,)), ((), ())),
                          preferred_element_type=jnp.float32)
    dist = nxc + c2_ref[...]
    m = jnp.min(dist, axis=1, keepdims=True)
    onehot = (dist <= m).astype(jnp.float32)
    qi = lax.dot_general(onehot, w2_ref[...], (((1,), (0,)), ((), ())),
                         preferred_element_type=jnp.float32)
    q = qi[:, :D]
    q_ref[...] = h + (q - h)
    idx_ref[...] = qi[:, D:D + 1].astype(jnp.int32)


@jax.jit
def kernel(h, codebook):
    bsz, t, d = h.shape
    n = bsz * t
    grid = n // TB
    flat = h.reshape(n, d)
    w1 = -2.0 * codebook.T
    c2 = jnp.sum(codebook ** 2, axis=1)[None, :]
    w2 = jnp.concatenate(
        [codebook, jnp.arange(K, dtype=jnp.float32)[:, None]], axis=1)
    q_flat, idx_col = pl.pallas_call(
        _vq_body,
        grid=(grid,),
        in_specs=[
            pl.BlockSpec((TB, d), lambda i: (i, 0)),
            pl.BlockSpec((d, K), lambda i: (0, 0)),
            pl.BlockSpec((1, K), lambda i: (0, 0)),
            pl.BlockSpec((K, d + 1), lambda i: (0, 0)),
        ],
        out_specs=[
            pl.BlockSpec((TB, d), lambda i: (i, 0)),
            pl.BlockSpec((TB, 1), lambda i: (i, 0)),
        ],
        out_shape=[
            jax.ShapeDtypeStruct((n, d), jnp.float32),
            jax.ShapeDtypeStruct((n, 1), jnp.int32),
        ],
    )(flat, w1, c2, w2)
    return q_flat.reshape(bsz, t, d), idx_col.reshape(bsz, t)
